# baseline (speedup 1.0000x reference)
import functools

import jax
import jax.numpy as jnp
from jax import lax
from jax.experimental import pallas as pl
from jax.experimental.pallas import tpu as pltpu
from jax.experimental.pallas import tpu_sc as plsc

_N = 10000
_E = 320000
_D = 128
_F = _D // 2

_CH = 128
_NPROC = 157
_EPAD = 16 * _NPROC * _CH
_WB = 200
_NWB = _N // _WB


def _matmul(x, W):
    def mm(x_ref, w_ref, o_ref):
        o_ref[...] = jnp.dot(x_ref[...], w_ref[...],
                             preferred_element_type=jnp.float32)

    return pl.pallas_call(
        mm,
        grid=(_N // 400,),
        in_specs=[
            pl.BlockSpec((400, _D), lambda i: (i, 0)),
            pl.BlockSpec((_D, _D), lambda i: (0, 0)),
        ],
        out_specs=pl.BlockSpec((400, _D), lambda i: (i, 0)),
        out_shape=jax.ShapeDtypeStruct((_N, _D), jnp.float32),
    )(x, W)


def _sc_aggregate(h2, src2, dst, wexp):
    mesh = plsc.VectorSubcoreMesh(core_axis_name="c", subcore_axis_name="s")

    @functools.partial(
        pl.kernel,
        out_type=jax.ShapeDtypeStruct((_N, 2, _F), jnp.float32),
        mesh=mesh,
        scratch_types=[
            pltpu.VMEM((_NPROC, _CH), jnp.int32),
            pltpu.VMEM((_NPROC, _CH), jnp.int32),
            pltpu.VMEM((_CH, 16), jnp.float32),
            pltpu.VMEM((_CH, _F), jnp.float32),
            pltpu.VMEM((_WB, _F), jnp.float32),
            pltpu.VMEM_SHARED((_N, _F), jnp.float32),
        ],
        compiler_params=pltpu.CompilerParams(use_tc_tiling_on_sc=False),
    )
    def k(h_hbm, src_hbm, dst_hbm, w_hbm, out_hbm,
          src_v, dst_v, w_v, rows_v, z_v, acc):
        c = lax.axis_index("c")
        s = lax.axis_index("s")

        pltpu.sync_copy(src_hbm.at[c, s], src_v)
        pltpu.sync_copy(dst_hbm.at[s], dst_v)

        zero16 = jnp.zeros((16,), jnp.float32)
        n_my_chunks = (_NWB - s + 15) // 16

        def zfill(j, carry):
            for d_ in range(_F // 16):
                z_v[j, pl.ds(d_ * 16, 16)] = zero16
            return carry

        lax.fori_loop(0, _WB, zfill, 0)

        def zcopy(i, carry):
            pltpu.sync_copy(z_v, acc.at[pl.ds((s + 16 * i) * _WB, _WB)])
            return carry

        lax.fori_loop(0, n_my_chunks, zcopy, 0)

        plsc.subcore_barrier()

        def body(i, carry):
            pltpu.sync_copy(h_hbm.at[src_v.at[i]], rows_v)
            pltpu.sync_copy(w_hbm.at[s, i], w_v)

            def scale(j, c2):
                wvec = w_v[j]
                for d_ in range(_F // 16):
                    sl = pl.ds(d_ * 16, 16)
                    rows_v[j, sl] = rows_v[j, sl] * wvec
                return c2

            lax.fori_loop(0, _CH, scale, 0)
            pltpu.sync_copy(rows_v, acc.at[dst_v.at[i]], add=True)
            return carry

        lax.fori_loop(0, _NPROC, body, 0)

        plsc.subcore_barrier()

        def wb(i, carry):
            r0 = (s + 16 * i) * _WB
            pltpu.sync_copy(acc.at[pl.ds(r0, _WB)], z_v)

            def rl(j, c2):
                for d_ in range(_F // 16):
                    sl = pl.ds(d_ * 16, 16)
                    z_v[j, sl] = jnp.maximum(z_v[j, sl], 0.0)
                return c2

            lax.fori_loop(0, _WB, rl, 0)
            pltpu.sync_copy(z_v, out_hbm.at[pl.ds(r0, _WB), c])
            return carry

        lax.fori_loop(0, n_my_chunks, wb, 0)

    return k(h2, src2, dst, wexp)


def kernel(x, edge_index, edge_weight, W):
    h = _matmul(x, W)
    h2 = h.reshape(2 * _N, _F)
    pad = _EPAD - _E
    src = jnp.pad(edge_index[1], (0, pad)).reshape(16, _NPROC, _CH)
    src2 = jnp.stack([src * 2, src * 2 + 1])
    dst = jnp.pad(edge_index[0], (0, pad)).reshape(16, _NPROC, _CH)
    wexp = jnp.broadcast_to(
        jnp.pad(edge_weight, (0, pad))[:, None], (_EPAD, 16)
    ).reshape(16, _NPROC, _CH, 16)
    out = _sc_aggregate(h2, src2, dst, wexp)
    return out.reshape(_N, _D)


# speedup vs baseline: 1.1349x; 1.1349x over previous
import functools

import jax
import jax.numpy as jnp
from jax import lax
from jax.experimental import pallas as pl
from jax.experimental.pallas import tpu as pltpu
from jax.experimental.pallas import tpu_sc as plsc

_N = 10000
_E = 320000
_D = 128
_F = _D // 2

_CH = 128
_NCHUNK = 157
_EPT = _CH * _NCHUNK
_EPAD = 16 * _EPT
_WB = 200
_NWB = _N // _WB


def _matmul(x, W):
    def mm(x_ref, w_ref, o_ref):
        o_ref[...] = jnp.dot(x_ref[...], w_ref[...],
                             preferred_element_type=jnp.float32)

    return pl.pallas_call(
        mm,
        grid=(_N // 400,),
        in_specs=[
            pl.BlockSpec((400, _D), lambda i: (i, 0)),
            pl.BlockSpec((_D, _D), lambda i: (0, 0)),
        ],
        out_specs=pl.BlockSpec((400, _D), lambda i: (i, 0)),
        out_shape=jax.ShapeDtypeStruct((_N, _D), jnp.float32),
    )(x, W)


def _sc_aggregate(h, src, dst, wexp):
    mesh = plsc.VectorSubcoreMesh(core_axis_name="c", subcore_axis_name="s")

    @functools.partial(
        pl.kernel,
        out_type=jax.ShapeDtypeStruct((_N, 2, _F), jnp.float32),
        mesh=mesh,
        scratch_types=[
            pltpu.VMEM((_NCHUNK, _CH), jnp.int32),
            pltpu.VMEM((1, _CH), jnp.int32),
            pltpu.VMEM((_CH, 16), jnp.float32),
            pltpu.VMEM((_CH, _F), jnp.float32),
            pltpu.VMEM((_WB, _F), jnp.float32),
            pltpu.VMEM_SHARED((_N, _F), jnp.float32),
            pltpu.VMEM_SHARED((_N, _F), jnp.float32),
            pltpu.SemaphoreType.DMA,
        ],
        compiler_params=pltpu.CompilerParams(use_tc_tiling_on_sc=False),
    )
    def k(h_hbm, src_hbm, dst_hbm, w_hbm, out_hbm,
          src_v, dst_v, wexp_v, rows_v, z_v, acc, h_sp, sem):
        c = lax.axis_index("c")
        s = lax.axis_index("s")

        pltpu.sync_copy(src_hbm.at[s], src_v)

        zero16 = jnp.zeros((16,), jnp.float32)
        n_my_chunks = (_NWB - s + 15) // 16

        def zfill(j, carry):
            for d_ in range(_F // 16):
                z_v[j, pl.ds(d_ * 16, 16)] = zero16
            return carry

        lax.fori_loop(0, _WB, zfill, 0)

        def zcopy(i, carry):
            r0 = (s + 16 * i) * _WB
            pltpu.sync_copy(
                h_hbm.at[pl.ds(r0, _WB), pl.ds(c * _F, _F)],
                h_sp.at[pl.ds(r0, _WB)])
            pltpu.sync_copy(z_v, acc.at[pl.ds(r0, _WB)])
            return carry

        lax.fori_loop(0, n_my_chunks, zcopy, 0)

        plsc.subcore_barrier()

        def chunk(i, carry):
            cp = pltpu.async_copy(h_sp.at[src_v.at[i]], rows_v, sem)
            pltpu.sync_copy(w_hbm.at[s, i], wexp_v)
            pltpu.sync_copy(dst_hbm.at[s, i], dst_v.at[0])
            cp.wait()

            def scale(j, c2):
                wvec = wexp_v[j]
                for d_ in range(_F // 16):
                    sl = pl.ds(d_ * 16, 16)
                    rows_v[j, sl] = rows_v[j, sl] * wvec
                return c2

            lax.fori_loop(0, _CH, scale, 0)
            pltpu.sync_copy(rows_v, acc.at[dst_v.at[0]], add=True)
            return carry

        lax.fori_loop(0, _NCHUNK, chunk, 0)

        plsc.subcore_barrier()

        def wb(i, carry):
            r0 = (s + 16 * i) * _WB
            pltpu.sync_copy(acc.at[pl.ds(r0, _WB)], z_v)

            def rl(j, c2):
                for d_ in range(_F // 16):
                    sl = pl.ds(d_ * 16, 16)
                    z_v[j, sl] = jnp.maximum(z_v[j, sl], 0.0)
                return c2

            lax.fori_loop(0, _WB, rl, 0)
            pltpu.sync_copy(z_v, out_hbm.at[pl.ds(r0, _WB), c])
            return carry

        lax.fori_loop(0, n_my_chunks, wb, 0)

    return k(h, src, dst, wexp)


def kernel(x, edge_index, edge_weight, W):
    h = _matmul(x, W)
    pad = _EPAD - _E
    src = jnp.pad(edge_index[1], (0, pad)).reshape(16, _NCHUNK, _CH)
    dst = jnp.pad(edge_index[0], (0, pad)).reshape(16, _NCHUNK, _CH)
    wexp = jnp.broadcast_to(
        jnp.pad(edge_weight, (0, pad))[:, None], (_EPAD, 16)
    ).reshape(16, _NCHUNK, _CH, 16)
    out = _sc_aggregate(h, src, dst, wexp)
    return out.reshape(_N, _D)
